# Initial kernel scaffold; baseline (speedup 1.0000x reference)
import jax
import jax.numpy as jnp
from jax.experimental import pallas as pl


def kernel(feat, offset, W1, b1, gamma, beta, rmean, rvar, W2, b2, W3, b3):
    raise NotImplementedError("write your pallas kernel here")


# speedup vs baseline: 9.9341x; 9.9341x over previous
import jax
import jax.numpy as jnp
from jax.experimental import pallas as pl
from jax.experimental.pallas import tpu as pltpu

_B = 16
_C = 256
_H = 256
_KOUT = 18
_BLK = 2048


def _seg_mlp_body(feat_ref, starts_ref, ends_ref, invc_ref,
                  w1_ref, bn1s_ref, bn1b_ref, w2_ref, b2_ref, w3_ref, b3_ref,
                  out_ref, acc_ref, *, nsteps):
    i = pl.program_id(0)

    @pl.when(i == 0)
    def _init():
        acc_ref[...] = jnp.zeros_like(acc_ref)

    rows = jax.lax.broadcasted_iota(jnp.int32, (_BLK, _B), 0) + i * _BLK
    starts = starts_ref[...]
    ends = ends_ref[...]
    onehot = ((rows >= starts) & (rows < ends)).astype(jnp.float32)
    acc_ref[...] += jax.lax.dot_general(
        onehot, feat_ref[...],
        dimension_numbers=(((0,), (0,)), ((), ())),
        preferred_element_type=jnp.float32)

    @pl.when(i == nsteps - 1)
    def _head():
        gf = acc_ref[...] * invc_ref[...]
        h = jnp.dot(gf, w1_ref[...], preferred_element_type=jnp.float32)
        h = h * bn1s_ref[...] + bn1b_ref[...]
        h = jnp.maximum(h, 0.0)
        h = jnp.dot(h, w2_ref[...], preferred_element_type=jnp.float32) + b2_ref[...]
        h = jnp.maximum(h, 0.0)
        out_ref[...] = (jnp.dot(h, w3_ref[...], preferred_element_type=jnp.float32)
                        + b3_ref[...])


def kernel(feat, offset, W1, b1, gamma, beta, rmean, rvar, W2, b2, W3, b3):
    n = feat.shape[0]
    nsteps = n // _BLK

    offset = offset.astype(jnp.int32)
    starts = jnp.concatenate([jnp.zeros((1,), jnp.int32), offset[:-1]])
    ends = offset
    counts = jnp.maximum((ends - starts).astype(jnp.float32), 1.0)
    invc = (1.0 / counts).reshape(_B, 1)
    bn1s = gamma * jax.lax.rsqrt(rvar + 1e-5)
    bn1b = (b1 - rmean) * bn1s + beta

    import functools
    body = functools.partial(_seg_mlp_body, nsteps=nsteps)
    out = pl.pallas_call(
        body,
        grid=(nsteps,),
        in_specs=[
            pl.BlockSpec((_BLK, _C), lambda i: (i, 0)),
            pl.BlockSpec((1, _B), lambda i: (0, 0)),
            pl.BlockSpec((1, _B), lambda i: (0, 0)),
            pl.BlockSpec((_B, 1), lambda i: (0, 0)),
            pl.BlockSpec((_C, _H), lambda i: (0, 0)),
            pl.BlockSpec((1, _H), lambda i: (0, 0)),
            pl.BlockSpec((1, _H), lambda i: (0, 0)),
            pl.BlockSpec((_H, _H), lambda i: (0, 0)),
            pl.BlockSpec((1, _H), lambda i: (0, 0)),
            pl.BlockSpec((_H, _KOUT), lambda i: (0, 0)),
            pl.BlockSpec((1, _KOUT), lambda i: (0, 0)),
        ],
        out_specs=pl.BlockSpec((_B, _KOUT), lambda i: (0, 0)),
        out_shape=jax.ShapeDtypeStruct((_B, _KOUT), jnp.float32),
        scratch_shapes=[pltpu.VMEM((_B, _C), jnp.float32)],
        compiler_params=pltpu.CompilerParams(
            dimension_semantics=("arbitrary",)),
    )(feat, starts.reshape(1, _B), ends.reshape(1, _B), invc,
      W1, bn1s.reshape(1, _H), bn1b.reshape(1, _H),
      W2, b2.reshape(1, _H), W3, b3.reshape(1, _KOUT))
    return out.reshape(_B, 6, 3)


# speedup vs baseline: 15.7752x; 1.5880x over previous
import jax
import jax.numpy as jnp
from jax.experimental import pallas as pl
from jax.experimental.pallas import tpu as pltpu

_B = 16
_C = 256
_H = 256
_KOUT = 18
_BLK = 8192


def _seg_mlp_body(feat_ref, starts_ref, ends_ref, invc_ref,
                  w1_ref, bn1s_ref, bn1b_ref, w2_ref, b2_ref, w3_ref, b3_ref,
                  out_ref, acc_ref, *, nsteps):
    i = pl.program_id(0)

    @pl.when(i == 0)
    def _init():
        acc_ref[...] = jnp.zeros_like(acc_ref)

    rows = jax.lax.broadcasted_iota(jnp.int32, (_B, _BLK), 1) + i * _BLK
    starts = starts_ref[...]
    ends = ends_ref[...]
    onehot_t = ((rows >= starts) & (rows < ends)).astype(jnp.float32)
    acc_ref[...] += jnp.dot(onehot_t, feat_ref[...],
                            preferred_element_type=jnp.float32)

    @pl.when(i == nsteps - 1)
    def _head():
        gf = acc_ref[...] * invc_ref[...]
        h = jnp.dot(gf, w1_ref[...], preferred_element_type=jnp.float32)
        h = h * bn1s_ref[...] + bn1b_ref[...]
        h = jnp.maximum(h, 0.0)
        h = jnp.dot(h, w2_ref[...], preferred_element_type=jnp.float32) + b2_ref[...]
        h = jnp.maximum(h, 0.0)
        out_ref[...] = (jnp.dot(h, w3_ref[...], preferred_element_type=jnp.float32)
                        + b3_ref[...])


def kernel(feat, offset, W1, b1, gamma, beta, rmean, rvar, W2, b2, W3, b3):
    n = feat.shape[0]
    nsteps = n // _BLK

    offset = offset.astype(jnp.int32)
    starts = jnp.concatenate([jnp.zeros((1,), jnp.int32), offset[:-1]])
    ends = offset
    counts = jnp.maximum((ends - starts).astype(jnp.float32), 1.0)
    invc = (1.0 / counts).reshape(_B, 1)
    bn1s = gamma * jax.lax.rsqrt(rvar + 1e-5)
    bn1b = (b1 - rmean) * bn1s + beta

    import functools
    body = functools.partial(_seg_mlp_body, nsteps=nsteps)
    out = pl.pallas_call(
        body,
        grid=(nsteps,),
        in_specs=[
            pl.BlockSpec((_BLK, _C), lambda i: (i, 0)),
            pl.BlockSpec((_B, 1), lambda i: (0, 0)),
            pl.BlockSpec((_B, 1), lambda i: (0, 0)),
            pl.BlockSpec((_B, 1), lambda i: (0, 0)),
            pl.BlockSpec((_C, _H), lambda i: (0, 0)),
            pl.BlockSpec((1, _H), lambda i: (0, 0)),
            pl.BlockSpec((1, _H), lambda i: (0, 0)),
            pl.BlockSpec((_H, _H), lambda i: (0, 0)),
            pl.BlockSpec((1, _H), lambda i: (0, 0)),
            pl.BlockSpec((_H, _KOUT), lambda i: (0, 0)),
            pl.BlockSpec((1, _KOUT), lambda i: (0, 0)),
        ],
        out_specs=pl.BlockSpec((_B, _KOUT), lambda i: (0, 0)),
        out_shape=jax.ShapeDtypeStruct((_B, _KOUT), jnp.float32),
        scratch_shapes=[pltpu.VMEM((_B, _C), jnp.float32)],
        compiler_params=pltpu.CompilerParams(
            dimension_semantics=("arbitrary",)),
    )(feat, starts.reshape(_B, 1), ends.reshape(_B, 1), invc,
      W1, bn1s.reshape(1, _H), bn1b.reshape(1, _H),
      W2, b2.reshape(1, _H), W3, b3.reshape(1, _KOUT))
    return out.reshape(_B, 6, 3)
